# baseline (speedup 1.0000x reference)
import dataclasses
import functools

import jax
import jax.numpy as jnp
from jax import lax
from jax.experimental import pallas as pl
from jax.experimental.pallas import tpu as pltpu
from jax.experimental.pallas import tpu_sc as plsc

_SC_CORES = 2
_SC_SUBCORES = 16
_SC_WORKERS = _SC_CORES * _SC_SUBCORES
_SC_LANES = 16

_TB = 128


def _lane_merge(m1, i1, m2, i2, n):
    big1 = jnp.min(m1, axis=1, keepdims=True)
    big_i1 = jnp.min(jnp.where(m1 == big1, i1, n), axis=1, keepdims=True)
    win = i1 == big_i1
    vals2 = jnp.where(win, m2, m1)
    idx2 = jnp.where(win, i2, i1)
    big2 = jnp.min(vals2, axis=1, keepdims=True)
    big_i2 = jnp.min(jnp.where(vals2 == big2, idx2, n), axis=1, keepdims=True)
    return big1, big_i1, big2, big_i2


def _top2_body(q_ref, k_ref, i1_ref, i2_ref, ga_ref, gc_ref,
               ks_m1, ks_i1, ks_m2, ks_i2, rp_m1, rp_i1, rp_m2, rp_i2):
    b = pl.program_id(0)
    rb = pl.program_id(1)
    n = k_ref.shape[2]
    nb = n // _TB
    inf = jnp.float32(jnp.inf)

    @pl.when(rb == 0)
    def _init():
        ks_m1[...] = jnp.full((n, _TB), inf)
        ks_m2[...] = jnp.full((n, _TB), inf)
        ks_i1[...] = jnp.full((n, _TB), n, jnp.int32)
        ks_i2[...] = jnp.full((n, _TB), n, jnp.int32)

    x_q = q_ref[0, :, 0:1]
    y_q = q_ref[0, :, 1:2]
    z_q = q_ref[0, :, 2:3]
    lane = lax.broadcasted_iota(jnp.int32, (_TB, _TB), 1)
    row = lax.broadcasted_iota(jnp.int32, (_TB, _TB), 0) + rb * _TB
    ciq = lane + rb * _TB

    def tile(c, carry):
        m1, i1, m2, i2 = carry
        x_k = k_ref[0, 0:1, pl.ds(c * _TB, _TB)]
        y_k = k_ref[0, 1:2, pl.ds(c * _TB, _TB)]
        z_k = k_ref[0, 2:3, pl.ds(c * _TB, _TB)]
        dx = x_q - x_k
        d2 = dx * dx
        dy = y_q - y_k
        d2 = d2 + dy * dy
        dz = z_q - z_k
        d2 = d2 + dz * dz
        v = jnp.sqrt(d2)
        ci = lane + c * _TB
        v = jnp.where(ci == row, jnp.float32(1e9), v)
        take1 = v < m1
        take2 = v < m2
        m2n = jnp.where(take2, v, m2)
        i2n = jnp.where(take2, ci, i2)
        m2 = jnp.where(take1, m1, m2n)
        i2 = jnp.where(take1, i1, i2n)
        m1 = jnp.where(take1, v, m1)
        i1 = jnp.where(take1, ci, i1)

        @pl.when(c > rb)
        def _mirror():
            vt = jnp.transpose(v)
            sl = pl.ds(c * _TB, _TB)
            km1 = ks_m1[sl, :]
            ki1 = ks_i1[sl, :]
            km2 = ks_m2[sl, :]
            ki2 = ks_i2[sl, :]
            kt1 = vt < km1
            kt2 = vt < km2
            km2n = jnp.where(kt2, vt, km2)
            ki2n = jnp.where(kt2, ciq, ki2)
            ks_m2[sl, :] = jnp.where(kt1, km1, km2n)
            ks_i2[sl, :] = jnp.where(kt1, ki1, ki2n)
            ks_m1[sl, :] = jnp.where(kt1, vt, km1)
            ks_i1[sl, :] = jnp.where(kt1, ciq, ki1)

        return m1, i1, m2, i2

    m1 = jnp.full((_TB, _TB), inf)
    m2 = jnp.full((_TB, _TB), inf)
    i1 = jnp.full((_TB, _TB), n, jnp.int32)
    i2 = jnp.full((_TB, _TB), n, jnp.int32)
    m1, i1, m2, i2 = lax.fori_loop(rb, nb, tile, (m1, i1, m2, i2))

    r1, ri1, r2, ri2 = _lane_merge(m1, i1, m2, i2, n)
    sl = pl.ds(rb * _TB, _TB)
    rp_m1[sl, :] = r1
    rp_i1[sl, :] = ri1
    rp_m2[sl, :] = r2
    rp_i2[sl, :] = ri2

    @pl.when(rb == nb - 1)
    def _finish():
        base = b * n
        for blk in range(nb):
            bsl = pl.ds(blk * _TB, _TB)
            kb1, kbi1, kb2, kbi2 = _lane_merge(
                ks_m1[bsl, :], ks_i1[bsl, :], ks_m2[bsl, :], ks_i2[bsl, :], n
            )
            rm1 = rp_m1[bsl, :]
            rj1 = rp_i1[bsl, :]
            rm2 = rp_m2[bsl, :]
            rj2 = rp_i2[bsl, :]
            kw = kb1 <= rm1
            fi1 = jnp.where(kw, kbi1, rj1)
            t = kb2 <= rm1
            u = kb1 <= rm2
            fi2 = jnp.where(kw, jnp.where(t, kbi2, rj1),
                            jnp.where(u, kbi1, rj2))
            i1_ref[0, bsl, :] = fi1
            i2_ref[0, bsl, :] = fi2
            ga_ref[0, bsl, :] = fi1 + base
            gc_ref[0, bsl, :] = fi2 + base


def _tc_top2(coords, coords_k):
    bsz, n, _ = coords.shape
    grid = (bsz, n // _TB)
    ispec = pl.BlockSpec((1, n, 1), lambda b, q: (b, 0, 0))
    ishape = jax.ShapeDtypeStruct((bsz, n, 1), jnp.int32)
    return pl.pallas_call(
        _top2_body,
        grid=grid,
        in_specs=[
            pl.BlockSpec((1, _TB, 3), lambda b, q: (b, q, 0)),
            pl.BlockSpec((1, 8, n), lambda b, q: (b, 0, 0)),
        ],
        out_specs=[ispec, ispec, ispec, ispec],
        out_shape=[ishape, ishape, ishape, ishape],
        scratch_shapes=[
            pltpu.VMEM((n, _TB), jnp.float32),
            pltpu.VMEM((n, _TB), jnp.int32),
            pltpu.VMEM((n, _TB), jnp.float32),
            pltpu.VMEM((n, _TB), jnp.int32),
            pltpu.VMEM((n, 1), jnp.float32),
            pltpu.VMEM((n, 1), jnp.int32),
            pltpu.VMEM((n, 1), jnp.float32),
            pltpu.VMEM((n, 1), jnp.int32),
        ],
    )(coords, coords_k)


def _sc_gather(table_t, idx_a, idx_c):
    rows = idx_a.shape[1]
    per_worker = rows * 128
    v = table_t.shape[1]
    mesh = plsc.VectorSubcoreMesh(core_axis_name="c", subcore_axis_name="s")
    out_t = jax.ShapeDtypeStruct((_SC_WORKERS, 3, per_worker), jnp.float32)

    cp = pltpu.CompilerParams()
    if "needs_layout_passes" in pltpu.CompilerParams.__dataclass_fields__:
        cp = dataclasses.replace(cp, needs_layout_passes=False)

    @functools.partial(
        pl.kernel,
        mesh=mesh,
        out_type=[out_t, out_t],
        scratch_types=[
            pltpu.VMEM((3, v), jnp.float32),
            pltpu.VMEM((rows, 128), jnp.int32),
            pltpu.VMEM((rows, 128), jnp.int32),
            pltpu.VMEM((3, per_worker), jnp.float32),
            pltpu.VMEM((3, per_worker), jnp.float32),
            pltpu.SemaphoreType.DMA,
        ],
        compiler_params=cp,
    )
    def gather_kernel(tab_hbm, ia_hbm, ic_hbm, oa_hbm, oc_hbm,
                      tab_v, ia_v, ic_v, ba_v, bc_v, sem):
        wid = lax.axis_index("s") * _SC_CORES + lax.axis_index("c")
        tab_cp = pltpu.async_copy(tab_hbm, tab_v, sem)
        pltpu.sync_copy(ia_hbm.at[wid], ia_v)
        pltpu.sync_copy(ic_hbm.at[wid], ic_v)
        tab_cp.wait()
        for idx_v, buf_v in ((ia_v, ba_v), (ic_v, bc_v)):
            for r in range(rows):
                for g in range(128 // _SC_LANES):
                    iv = idx_v[r, pl.ds(g * _SC_LANES, _SC_LANES)]
                    pos = r * 128 + g * _SC_LANES
                    for c in range(3):
                        cv = jnp.full((_SC_LANES,), c, jnp.int32)
                        vals = plsc.load_gather(tab_v, [cv, iv])
                        buf_v[c, pl.ds(pos, _SC_LANES)] = vals
        pltpu.sync_copy(ba_v, oa_hbm.at[wid])
        pltpu.sync_copy(bc_v, oc_hbm.at[wid])

    return gather_kernel(table_t, idx_a, idx_c)


def kernel(coords, mask):
    del mask
    bsz, n, _ = coords.shape
    coords_k = jnp.transpose(
        jnp.pad(coords, ((0, 0), (0, 0), (0, 5))), (0, 2, 1)
    )
    i1, i2, ga, gc = _tc_top2(coords, coords_k)
    table_t = jnp.transpose(coords.reshape(bsz * n, 3))
    rows = (bsz * n) // (_SC_WORKERS * 128)
    out_a, out_c = _sc_gather(
        table_t,
        ga.reshape(_SC_WORKERS, rows, 128),
        gc.reshape(_SC_WORKERS, rows, 128),
    )
    a = jnp.transpose(out_a, (0, 2, 1)).reshape(bsz, n, 3)
    c = jnp.transpose(out_c, (0, 2, 1)).reshape(bsz, n, 3)
    return a, c, i1.reshape(bsz, n), i2.reshape(bsz, n)


# speedup vs baseline: 1.3168x; 1.3168x over previous
import dataclasses
import functools

import jax
import jax.numpy as jnp
from jax import lax
from jax.experimental import pallas as pl
from jax.experimental.pallas import tpu as pltpu
from jax.experimental.pallas import tpu_sc as plsc

_SC_CORES = 2
_SC_SUBCORES = 16
_SC_WORKERS = _SC_CORES * _SC_SUBCORES
_SC_LANES = 16

_TB = 256


def _lane_merge(m1, i1, m2, i2, n):
    big1 = jnp.min(m1, axis=1, keepdims=True)
    big_i1 = jnp.min(jnp.where(m1 == big1, i1, n), axis=1, keepdims=True)
    win = i1 == big_i1
    vals2 = jnp.where(win, m2, m1)
    idx2 = jnp.where(win, i2, i1)
    big2 = jnp.min(vals2, axis=1, keepdims=True)
    big_i2 = jnp.min(jnp.where(vals2 == big2, idx2, n), axis=1, keepdims=True)
    return big1, big_i1, big2, big_i2


def _top2_body(q_ref, k_ref, i1_ref, i2_ref, ga_ref, gc_ref,
               ks_m1, ks_i1, ks_m2, ks_i2, rp_m1, rp_i1, rp_m2, rp_i2):
    b = pl.program_id(0)
    rb = pl.program_id(1)
    n = k_ref.shape[2]
    nb = n // _TB
    inf = jnp.float32(jnp.inf)

    @pl.when(rb == 0)
    def _init():
        ks_m1[...] = jnp.full((n, _TB), inf)
        ks_m2[...] = jnp.full((n, _TB), inf)
        ks_i1[...] = jnp.full((n, _TB), n, jnp.int32)
        ks_i2[...] = jnp.full((n, _TB), n, jnp.int32)

    x_q = q_ref[0, :, 0:1]
    y_q = q_ref[0, :, 1:2]
    z_q = q_ref[0, :, 2:3]
    lane = lax.broadcasted_iota(jnp.int32, (_TB, _TB), 1)
    row = lax.broadcasted_iota(jnp.int32, (_TB, _TB), 0) + rb * _TB
    ciq = lane + rb * _TB

    def tile(c, carry):
        m1, i1, m2, i2 = carry
        x_k = k_ref[0, 0:1, pl.ds(c * _TB, _TB)]
        y_k = k_ref[0, 1:2, pl.ds(c * _TB, _TB)]
        z_k = k_ref[0, 2:3, pl.ds(c * _TB, _TB)]
        dx = x_q - x_k
        d2 = dx * dx
        dy = y_q - y_k
        d2 = d2 + dy * dy
        dz = z_q - z_k
        d2 = d2 + dz * dz
        v = jnp.sqrt(d2)
        ci = lane + c * _TB
        v = jnp.where(ci == row, jnp.float32(1e9), v)
        take1 = v < m1
        take2 = v < m2
        m2n = jnp.where(take2, v, m2)
        i2n = jnp.where(take2, ci, i2)
        m2 = jnp.where(take1, m1, m2n)
        i2 = jnp.where(take1, i1, i2n)
        m1 = jnp.where(take1, v, m1)
        i1 = jnp.where(take1, ci, i1)

        @pl.when(c > rb)
        def _mirror():
            vt = jnp.transpose(v)
            sl = pl.ds(c * _TB, _TB)
            km1 = ks_m1[sl, :]
            ki1 = ks_i1[sl, :]
            km2 = ks_m2[sl, :]
            ki2 = ks_i2[sl, :]
            kt1 = vt < km1
            kt2 = vt < km2
            km2n = jnp.where(kt2, vt, km2)
            ki2n = jnp.where(kt2, ciq, ki2)
            ks_m2[sl, :] = jnp.where(kt1, km1, km2n)
            ks_i2[sl, :] = jnp.where(kt1, ki1, ki2n)
            ks_m1[sl, :] = jnp.where(kt1, vt, km1)
            ks_i1[sl, :] = jnp.where(kt1, ciq, ki1)

        return m1, i1, m2, i2

    m1 = jnp.full((_TB, _TB), inf)
    m2 = jnp.full((_TB, _TB), inf)
    i1 = jnp.full((_TB, _TB), n, jnp.int32)
    i2 = jnp.full((_TB, _TB), n, jnp.int32)
    m1, i1, m2, i2 = lax.fori_loop(rb, nb, tile, (m1, i1, m2, i2))

    r1, ri1, r2, ri2 = _lane_merge(m1, i1, m2, i2, n)
    sl = pl.ds(rb * _TB, _TB)
    rp_m1[sl, :] = r1
    rp_i1[sl, :] = ri1
    rp_m2[sl, :] = r2
    rp_i2[sl, :] = ri2

    @pl.when(rb == nb - 1)
    def _finish():
        base = b * n
        for blk in range(nb):
            bsl = pl.ds(blk * _TB, _TB)
            kb1, kbi1, kb2, kbi2 = _lane_merge(
                ks_m1[bsl, :], ks_i1[bsl, :], ks_m2[bsl, :], ks_i2[bsl, :], n
            )
            rm1 = rp_m1[bsl, :]
            rj1 = rp_i1[bsl, :]
            rm2 = rp_m2[bsl, :]
            rj2 = rp_i2[bsl, :]
            kw = kb1 <= rm1
            fi1 = jnp.where(kw, kbi1, rj1)
            t = kb2 <= rm1
            u = kb1 <= rm2
            fi2 = jnp.where(kw, jnp.where(t, kbi2, rj1),
                            jnp.where(u, kbi1, rj2))
            i1_ref[0, bsl, :] = fi1
            i2_ref[0, bsl, :] = fi2
            ga_ref[0, bsl, :] = fi1 + base
            gc_ref[0, bsl, :] = fi2 + base


def _tc_top2(coords, coords_k):
    bsz, n, _ = coords.shape
    grid = (bsz, n // _TB)
    ispec = pl.BlockSpec((1, n, 1), lambda b, q: (b, 0, 0))
    ishape = jax.ShapeDtypeStruct((bsz, n, 1), jnp.int32)
    return pl.pallas_call(
        _top2_body,
        grid=grid,
        in_specs=[
            pl.BlockSpec((1, _TB, 3), lambda b, q: (b, q, 0)),
            pl.BlockSpec((1, 8, n), lambda b, q: (b, 0, 0)),
        ],
        out_specs=[ispec, ispec, ispec, ispec],
        out_shape=[ishape, ishape, ishape, ishape],
        scratch_shapes=[
            pltpu.VMEM((n, _TB), jnp.float32),
            pltpu.VMEM((n, _TB), jnp.int32),
            pltpu.VMEM((n, _TB), jnp.float32),
            pltpu.VMEM((n, _TB), jnp.int32),
            pltpu.VMEM((n, 1), jnp.float32),
            pltpu.VMEM((n, 1), jnp.int32),
            pltpu.VMEM((n, 1), jnp.float32),
            pltpu.VMEM((n, 1), jnp.int32),
        ],
    )(coords, coords_k)


def _sc_gather(table_t, idx_a, idx_c):
    rows = idx_a.shape[1]
    per_worker = rows * 128
    v = table_t.shape[1]
    mesh = plsc.VectorSubcoreMesh(core_axis_name="c", subcore_axis_name="s")
    out_t = jax.ShapeDtypeStruct((_SC_WORKERS, 3, per_worker), jnp.float32)

    cp = pltpu.CompilerParams()
    if "needs_layout_passes" in pltpu.CompilerParams.__dataclass_fields__:
        cp = dataclasses.replace(cp, needs_layout_passes=False)

    @functools.partial(
        pl.kernel,
        mesh=mesh,
        out_type=[out_t, out_t],
        scratch_types=[
            pltpu.VMEM((3, v), jnp.float32),
            pltpu.VMEM((rows, 128), jnp.int32),
            pltpu.VMEM((rows, 128), jnp.int32),
            pltpu.VMEM((3, per_worker), jnp.float32),
            pltpu.VMEM((3, per_worker), jnp.float32),
            pltpu.SemaphoreType.DMA,
        ],
        compiler_params=cp,
    )
    def gather_kernel(tab_hbm, ia_hbm, ic_hbm, oa_hbm, oc_hbm,
                      tab_v, ia_v, ic_v, ba_v, bc_v, sem):
        wid = lax.axis_index("s") * _SC_CORES + lax.axis_index("c")
        tab_cp = pltpu.async_copy(tab_hbm, tab_v, sem)
        pltpu.sync_copy(ia_hbm.at[wid], ia_v)
        pltpu.sync_copy(ic_hbm.at[wid], ic_v)
        tab_cp.wait()
        for idx_v, buf_v in ((ia_v, ba_v), (ic_v, bc_v)):
            for r in range(rows):
                for g in range(128 // _SC_LANES):
                    iv = idx_v[r, pl.ds(g * _SC_LANES, _SC_LANES)]
                    pos = r * 128 + g * _SC_LANES
                    for c in range(3):
                        cv = jnp.full((_SC_LANES,), c, jnp.int32)
                        vals = plsc.load_gather(tab_v, [cv, iv])
                        buf_v[c, pl.ds(pos, _SC_LANES)] = vals
        pltpu.sync_copy(ba_v, oa_hbm.at[wid])
        pltpu.sync_copy(bc_v, oc_hbm.at[wid])

    return gather_kernel(table_t, idx_a, idx_c)


def kernel(coords, mask):
    del mask
    bsz, n, _ = coords.shape
    coords_k = jnp.transpose(
        jnp.pad(coords, ((0, 0), (0, 0), (0, 5))), (0, 2, 1)
    )
    i1, i2, ga, gc = _tc_top2(coords, coords_k)
    table_t = jnp.transpose(coords.reshape(bsz * n, 3))
    rows = (bsz * n) // (_SC_WORKERS * 128)
    out_a, out_c = _sc_gather(
        table_t,
        ga.reshape(_SC_WORKERS, rows, 128),
        gc.reshape(_SC_WORKERS, rows, 128),
    )
    a = jnp.transpose(out_a, (0, 2, 1)).reshape(bsz, n, 3)
    c = jnp.transpose(out_c, (0, 2, 1)).reshape(bsz, n, 3)
    return a, c, i1.reshape(bsz, n), i2.reshape(bsz, n)


# speedup vs baseline: 2.0669x; 1.5696x over previous
import dataclasses
import functools

import jax
import jax.numpy as jnp
from jax import lax
from jax.experimental import pallas as pl
from jax.experimental.pallas import tpu as pltpu
from jax.experimental.pallas import tpu_sc as plsc

_SC_CORES = 2
_SC_SUBCORES = 16
_SC_WORKERS = _SC_CORES * _SC_SUBCORES
_SC_LANES = 16

_QB = 512
_KC = 128


def _top2_body(q_ref, k_ref, i1_ref, i2_ref, ga_ref, gc_ref):
    b = pl.program_id(0)
    qb = pl.program_id(1)
    n = k_ref.shape[2]
    x_q = q_ref[0, :, 0:1]
    y_q = q_ref[0, :, 1:2]
    z_q = q_ref[0, :, 2:3]

    lane = lax.broadcasted_iota(jnp.int32, (_QB, _KC), 1)
    row = lax.broadcasted_iota(jnp.int32, (_QB, _KC), 0) + qb * _QB
    inf = jnp.float32(jnp.inf)
    m1 = jnp.full((_QB, _KC), inf)
    m2 = jnp.full((_QB, _KC), inf)
    i1 = jnp.full((_QB, _KC), n, jnp.int32)
    i2 = jnp.full((_QB, _KC), n, jnp.int32)
    for c in range(n // _KC):
        x_k = k_ref[0, 0:1, pl.ds(c * _KC, _KC)]
        y_k = k_ref[0, 1:2, pl.ds(c * _KC, _KC)]
        z_k = k_ref[0, 2:3, pl.ds(c * _KC, _KC)]
        dx = x_q - x_k
        d2 = dx * dx
        dy = y_q - y_k
        d2 = d2 + dy * dy
        dz = z_q - z_k
        d2 = d2 + dz * dz
        v = jnp.sqrt(d2)
        ci = lane + c * _KC
        v = jnp.where(ci == row, jnp.float32(1e9), v)
        take1 = v < m1
        take2 = v < m2
        m2n = jnp.where(take2, v, m2)
        i2n = jnp.where(take2, ci, i2)
        m2 = jnp.where(take1, m1, m2n)
        i2 = jnp.where(take1, i1, i2n)
        m1 = jnp.where(take1, v, m1)
        i1 = jnp.where(take1, ci, i1)

    big1 = jnp.min(m1, axis=1, keepdims=True)
    big_i1 = jnp.min(jnp.where(m1 == big1, i1, n), axis=1, keepdims=True)
    win = i1 == big_i1
    vals2 = jnp.where(win, m2, m1)
    idx2 = jnp.where(win, i2, i1)
    big2 = jnp.min(vals2, axis=1, keepdims=True)
    big_i2 = jnp.min(jnp.where(vals2 == big2, idx2, n), axis=1, keepdims=True)

    i1_ref[0] = big_i1
    i2_ref[0] = big_i2
    base = b * n
    ga_ref[0] = big_i1 + base
    gc_ref[0] = big_i2 + base


def _tc_top2(coords, coords_k):
    bsz, n, _ = coords.shape
    grid = (bsz, n // _QB)
    ispec = pl.BlockSpec((1, _QB, 1), lambda b, q: (b, q, 0))
    ishape = jax.ShapeDtypeStruct((bsz, n, 1), jnp.int32)
    return pl.pallas_call(
        _top2_body,
        grid=grid,
        in_specs=[
            pl.BlockSpec((1, _QB, 3), lambda b, q: (b, q, 0)),
            pl.BlockSpec((1, 8, n), lambda b, q: (b, 0, 0)),
        ],
        out_specs=[ispec, ispec, ispec, ispec],
        out_shape=[ishape, ishape, ishape, ishape],
    )(coords, coords_k)


def _sc_gather(table_t, idx_a, idx_c):
    rows = idx_a.shape[1]
    per_worker = rows * 128
    v = table_t.shape[1]
    mesh = plsc.VectorSubcoreMesh(core_axis_name="c", subcore_axis_name="s")
    out_t = jax.ShapeDtypeStruct((_SC_WORKERS, 3, per_worker), jnp.float32)

    cp = pltpu.CompilerParams()
    if "needs_layout_passes" in pltpu.CompilerParams.__dataclass_fields__:
        cp = dataclasses.replace(cp, needs_layout_passes=False)

    @functools.partial(
        pl.kernel,
        mesh=mesh,
        out_type=[out_t, out_t],
        scratch_types=[
            pltpu.VMEM((3, v), jnp.float32),
            pltpu.VMEM((rows, 128), jnp.int32),
            pltpu.VMEM((rows, 128), jnp.int32),
            pltpu.VMEM((3, per_worker), jnp.float32),
            pltpu.VMEM((3, per_worker), jnp.float32),
            pltpu.SemaphoreType.DMA,
        ],
        compiler_params=cp,
    )
    def gather_kernel(tab_hbm, ia_hbm, ic_hbm, oa_hbm, oc_hbm,
                      tab_v, ia_v, ic_v, ba_v, bc_v, sem):
        wid = lax.axis_index("s") * _SC_CORES + lax.axis_index("c")
        tab_cp = pltpu.async_copy(tab_hbm, tab_v, sem)
        pltpu.sync_copy(ia_hbm.at[wid], ia_v)
        pltpu.sync_copy(ic_hbm.at[wid], ic_v)
        tab_cp.wait()
        for idx_v, buf_v in ((ia_v, ba_v), (ic_v, bc_v)):
            for r in range(rows):
                for g in range(128 // _SC_LANES):
                    iv = idx_v[r, pl.ds(g * _SC_LANES, _SC_LANES)]
                    pos = r * 128 + g * _SC_LANES
                    for c in range(3):
                        cv = jnp.full((_SC_LANES,), c, jnp.int32)
                        vals = plsc.load_gather(tab_v, [cv, iv])
                        buf_v[c, pl.ds(pos, _SC_LANES)] = vals
        pltpu.sync_copy(ba_v, oa_hbm.at[wid])
        pltpu.sync_copy(bc_v, oc_hbm.at[wid])

    return gather_kernel(table_t, idx_a, idx_c)


def kernel(coords, mask):
    del mask
    bsz, n, _ = coords.shape
    coords_k = jnp.transpose(
        jnp.pad(coords, ((0, 0), (0, 0), (0, 5))), (0, 2, 1)
    )
    i1, i2, ga, gc = _tc_top2(coords, coords_k)
    table_t = jnp.transpose(coords.reshape(bsz * n, 3))
    rows = (bsz * n) // (_SC_WORKERS * 128)
    out_a, out_c = _sc_gather(
        table_t,
        ga.reshape(_SC_WORKERS, rows, 128),
        gc.reshape(_SC_WORKERS, rows, 128),
    )
    a = jnp.transpose(out_a, (0, 2, 1)).reshape(bsz, n, 3)
    c = jnp.transpose(out_c, (0, 2, 1)).reshape(bsz, n, 3)
    return a, c, i1.reshape(bsz, n), i2.reshape(bsz, n)


# speedup vs baseline: 2.1032x; 1.0176x over previous
import dataclasses
import functools

import jax
import jax.numpy as jnp
from jax import lax
from jax.experimental import pallas as pl
from jax.experimental.pallas import tpu as pltpu
from jax.experimental.pallas import tpu_sc as plsc

_SC_CORES = 2
_SC_SUBCORES = 16
_SC_WORKERS = _SC_CORES * _SC_SUBCORES
_SC_LANES = 16

_QB = 1024
_KC = 128


def _top2_body(q_ref, k_ref, i1_ref, i2_ref, ga_ref, gc_ref):
    b = pl.program_id(0)
    qb = pl.program_id(1)
    n = k_ref.shape[2]
    x_q = q_ref[0, :, 0:1]
    y_q = q_ref[0, :, 1:2]
    z_q = q_ref[0, :, 2:3]

    lane = lax.broadcasted_iota(jnp.int32, (_QB, _KC), 1)
    row = lax.broadcasted_iota(jnp.int32, (_QB, _KC), 0) + qb * _QB
    inf = jnp.float32(jnp.inf)
    m1 = jnp.full((_QB, _KC), inf)
    m2 = jnp.full((_QB, _KC), inf)
    i1 = jnp.full((_QB, _KC), n, jnp.int32)
    i2 = jnp.full((_QB, _KC), n, jnp.int32)
    for c in range(n // _KC):
        x_k = k_ref[0, 0:1, pl.ds(c * _KC, _KC)]
        y_k = k_ref[0, 1:2, pl.ds(c * _KC, _KC)]
        z_k = k_ref[0, 2:3, pl.ds(c * _KC, _KC)]
        dx = x_q - x_k
        d2 = dx * dx
        dy = y_q - y_k
        d2 = d2 + dy * dy
        dz = z_q - z_k
        d2 = d2 + dz * dz
        v = jnp.sqrt(d2)
        ci = lane + c * _KC
        v = jnp.where(ci == row, jnp.float32(1e9), v)
        take1 = v < m1
        take2 = v < m2
        m2n = jnp.where(take2, v, m2)
        i2n = jnp.where(take2, ci, i2)
        m2 = jnp.where(take1, m1, m2n)
        i2 = jnp.where(take1, i1, i2n)
        m1 = jnp.where(take1, v, m1)
        i1 = jnp.where(take1, ci, i1)

    big1 = jnp.min(m1, axis=1, keepdims=True)
    big_i1 = jnp.min(jnp.where(m1 == big1, i1, n), axis=1, keepdims=True)
    win = i1 == big_i1
    vals2 = jnp.where(win, m2, m1)
    idx2 = jnp.where(win, i2, i1)
    big2 = jnp.min(vals2, axis=1, keepdims=True)
    big_i2 = jnp.min(jnp.where(vals2 == big2, idx2, n), axis=1, keepdims=True)

    i1_ref[0] = big_i1
    i2_ref[0] = big_i2
    base = b * n
    ga_ref[0] = big_i1 + base
    gc_ref[0] = big_i2 + base


def _tc_top2(coords, coords_k):
    bsz, n, _ = coords.shape
    grid = (bsz, n // _QB)
    ispec = pl.BlockSpec((1, _QB, 1), lambda b, q: (b, q, 0))
    ishape = jax.ShapeDtypeStruct((bsz, n, 1), jnp.int32)
    return pl.pallas_call(
        _top2_body,
        grid=grid,
        in_specs=[
            pl.BlockSpec((1, _QB, 3), lambda b, q: (b, q, 0)),
            pl.BlockSpec((1, 8, n), lambda b, q: (b, 0, 0)),
        ],
        out_specs=[ispec, ispec, ispec, ispec],
        out_shape=[ishape, ishape, ishape, ishape],
    )(coords, coords_k)


def _sc_gather(table_t, idx_a, idx_c):
    rows = idx_a.shape[1]
    per_worker = rows * 128
    v = table_t.shape[1]
    mesh = plsc.VectorSubcoreMesh(core_axis_name="c", subcore_axis_name="s")
    out_t = jax.ShapeDtypeStruct((_SC_WORKERS, 3, per_worker), jnp.float32)

    cp = pltpu.CompilerParams()
    if "needs_layout_passes" in pltpu.CompilerParams.__dataclass_fields__:
        cp = dataclasses.replace(cp, needs_layout_passes=False)

    @functools.partial(
        pl.kernel,
        mesh=mesh,
        out_type=[out_t, out_t],
        scratch_types=[
            pltpu.VMEM((3, v), jnp.float32),
            pltpu.VMEM((rows, 128), jnp.int32),
            pltpu.VMEM((rows, 128), jnp.int32),
            pltpu.VMEM((3, per_worker), jnp.float32),
            pltpu.VMEM((3, per_worker), jnp.float32),
            pltpu.SemaphoreType.DMA,
        ],
        compiler_params=cp,
    )
    def gather_kernel(tab_hbm, ia_hbm, ic_hbm, oa_hbm, oc_hbm,
                      tab_v, ia_v, ic_v, ba_v, bc_v, sem):
        wid = lax.axis_index("s") * _SC_CORES + lax.axis_index("c")
        tab_cp = pltpu.async_copy(tab_hbm, tab_v, sem)
        pltpu.sync_copy(ia_hbm.at[wid], ia_v)
        pltpu.sync_copy(ic_hbm.at[wid], ic_v)
        tab_cp.wait()
        for idx_v, buf_v in ((ia_v, ba_v), (ic_v, bc_v)):
            for r in range(rows):
                for g in range(128 // _SC_LANES):
                    iv = idx_v[r, pl.ds(g * _SC_LANES, _SC_LANES)]
                    pos = r * 128 + g * _SC_LANES
                    for c in range(3):
                        cv = jnp.full((_SC_LANES,), c, jnp.int32)
                        vals = plsc.load_gather(tab_v, [cv, iv])
                        buf_v[c, pl.ds(pos, _SC_LANES)] = vals
        pltpu.sync_copy(ba_v, oa_hbm.at[wid])
        pltpu.sync_copy(bc_v, oc_hbm.at[wid])

    return gather_kernel(table_t, idx_a, idx_c)


def kernel(coords, mask):
    del mask
    bsz, n, _ = coords.shape
    coords_k = jnp.transpose(
        jnp.pad(coords, ((0, 0), (0, 0), (0, 5))), (0, 2, 1)
    )
    i1, i2, ga, gc = _tc_top2(coords, coords_k)
    table_t = jnp.transpose(coords.reshape(bsz * n, 3))
    rows = (bsz * n) // (_SC_WORKERS * 128)
    out_a, out_c = _sc_gather(
        table_t,
        ga.reshape(_SC_WORKERS, rows, 128),
        gc.reshape(_SC_WORKERS, rows, 128),
    )
    a = jnp.transpose(out_a, (0, 2, 1)).reshape(bsz, n, 3)
    c = jnp.transpose(out_c, (0, 2, 1)).reshape(bsz, n, 3)
    return a, c, i1.reshape(bsz, n), i2.reshape(bsz, n)


# speedup vs baseline: 2.1725x; 1.0329x over previous
import dataclasses
import functools

import jax
import jax.numpy as jnp
from jax import lax
from jax.experimental import pallas as pl
from jax.experimental.pallas import tpu as pltpu
from jax.experimental.pallas import tpu_sc as plsc

_SC_CORES = 2
_SC_SUBCORES = 16
_SC_WORKERS = _SC_CORES * _SC_SUBCORES
_SC_LANES = 16

_QB = 2048
_KC = 128


def _top2_body(q_ref, k_ref, i1_ref, i2_ref, ga_ref, gc_ref):
    b = pl.program_id(0)
    qb = pl.program_id(1)
    n = k_ref.shape[2]
    x_q = q_ref[0, :, 0:1]
    y_q = q_ref[0, :, 1:2]
    z_q = q_ref[0, :, 2:3]

    lane = lax.broadcasted_iota(jnp.int32, (_QB, _KC), 1)
    row = lax.broadcasted_iota(jnp.int32, (_QB, _KC), 0) + qb * _QB
    inf = jnp.float32(jnp.inf)
    m1 = jnp.full((_QB, _KC), inf)
    m2 = jnp.full((_QB, _KC), inf)
    i1 = jnp.full((_QB, _KC), n, jnp.int32)
    i2 = jnp.full((_QB, _KC), n, jnp.int32)
    for c in range(n // _KC):
        x_k = k_ref[0, 0:1, pl.ds(c * _KC, _KC)]
        y_k = k_ref[0, 1:2, pl.ds(c * _KC, _KC)]
        z_k = k_ref[0, 2:3, pl.ds(c * _KC, _KC)]
        dx = x_q - x_k
        d2 = dx * dx
        dy = y_q - y_k
        d2 = d2 + dy * dy
        dz = z_q - z_k
        d2 = d2 + dz * dz
        v = jnp.sqrt(d2)
        ci = lane + c * _KC
        v = jnp.where(ci == row, jnp.float32(1e9), v)
        take1 = v < m1
        take2 = v < m2
        m2n = jnp.where(take2, v, m2)
        i2n = jnp.where(take2, ci, i2)
        m2 = jnp.where(take1, m1, m2n)
        i2 = jnp.where(take1, i1, i2n)
        m1 = jnp.where(take1, v, m1)
        i1 = jnp.where(take1, ci, i1)

    big1 = jnp.min(m1, axis=1, keepdims=True)
    big_i1 = jnp.min(jnp.where(m1 == big1, i1, n), axis=1, keepdims=True)
    win = i1 == big_i1
    vals2 = jnp.where(win, m2, m1)
    idx2 = jnp.where(win, i2, i1)
    big2 = jnp.min(vals2, axis=1, keepdims=True)
    big_i2 = jnp.min(jnp.where(vals2 == big2, idx2, n), axis=1, keepdims=True)

    i1_ref[0] = big_i1
    i2_ref[0] = big_i2
    base = b * n
    ga_ref[0] = big_i1 + base
    gc_ref[0] = big_i2 + base


def _tc_top2(coords, coords_k):
    bsz, n, _ = coords.shape
    grid = (bsz, n // _QB)
    ispec = pl.BlockSpec((1, _QB, 1), lambda b, q: (b, q, 0))
    ishape = jax.ShapeDtypeStruct((bsz, n, 1), jnp.int32)
    return pl.pallas_call(
        _top2_body,
        grid=grid,
        in_specs=[
            pl.BlockSpec((1, _QB, 3), lambda b, q: (b, q, 0)),
            pl.BlockSpec((1, 8, n), lambda b, q: (b, 0, 0)),
        ],
        out_specs=[ispec, ispec, ispec, ispec],
        out_shape=[ishape, ishape, ishape, ishape],
    )(coords, coords_k)


def _sc_gather(table_t, idx_a, idx_c):
    rows = idx_a.shape[1]
    per_worker = rows * 128
    v = table_t.shape[1]
    mesh = plsc.VectorSubcoreMesh(core_axis_name="c", subcore_axis_name="s")
    out_t = jax.ShapeDtypeStruct((_SC_WORKERS, 3, per_worker), jnp.float32)

    cp = pltpu.CompilerParams()
    if "needs_layout_passes" in pltpu.CompilerParams.__dataclass_fields__:
        cp = dataclasses.replace(cp, needs_layout_passes=False)

    @functools.partial(
        pl.kernel,
        mesh=mesh,
        out_type=[out_t, out_t],
        scratch_types=[
            pltpu.VMEM((3, v), jnp.float32),
            pltpu.VMEM((rows, 128), jnp.int32),
            pltpu.VMEM((rows, 128), jnp.int32),
            pltpu.VMEM((3, per_worker), jnp.float32),
            pltpu.VMEM((3, per_worker), jnp.float32),
            pltpu.SemaphoreType.DMA,
        ],
        compiler_params=cp,
    )
    def gather_kernel(tab_hbm, ia_hbm, ic_hbm, oa_hbm, oc_hbm,
                      tab_v, ia_v, ic_v, ba_v, bc_v, sem):
        wid = lax.axis_index("s") * _SC_CORES + lax.axis_index("c")
        tab_cp = pltpu.async_copy(tab_hbm, tab_v, sem)
        pltpu.sync_copy(ia_hbm.at[wid], ia_v)
        pltpu.sync_copy(ic_hbm.at[wid], ic_v)
        tab_cp.wait()
        for idx_v, buf_v in ((ia_v, ba_v), (ic_v, bc_v)):
            for r in range(rows):
                for g in range(128 // _SC_LANES):
                    iv = idx_v[r, pl.ds(g * _SC_LANES, _SC_LANES)]
                    pos = r * 128 + g * _SC_LANES
                    for c in range(3):
                        cv = jnp.full((_SC_LANES,), c, jnp.int32)
                        vals = plsc.load_gather(tab_v, [cv, iv])
                        buf_v[c, pl.ds(pos, _SC_LANES)] = vals
        pltpu.sync_copy(ba_v, oa_hbm.at[wid])
        pltpu.sync_copy(bc_v, oc_hbm.at[wid])

    return gather_kernel(table_t, idx_a, idx_c)


def kernel(coords, mask):
    del mask
    bsz, n, _ = coords.shape
    coords_k = jnp.transpose(
        jnp.pad(coords, ((0, 0), (0, 0), (0, 5))), (0, 2, 1)
    )
    i1, i2, ga, gc = _tc_top2(coords, coords_k)
    table_t = jnp.transpose(coords.reshape(bsz * n, 3))
    rows = (bsz * n) // (_SC_WORKERS * 128)
    out_a, out_c = _sc_gather(
        table_t,
        ga.reshape(_SC_WORKERS, rows, 128),
        gc.reshape(_SC_WORKERS, rows, 128),
    )
    a = jnp.transpose(out_a, (0, 2, 1)).reshape(bsz, n, 3)
    c = jnp.transpose(out_c, (0, 2, 1)).reshape(bsz, n, 3)
    return a, c, i1.reshape(bsz, n), i2.reshape(bsz, n)


# speedup vs baseline: 2.1783x; 1.0027x over previous
import dataclasses
import functools

import jax
import jax.numpy as jnp
from jax import lax
from jax.experimental import pallas as pl
from jax.experimental.pallas import tpu as pltpu
from jax.experimental.pallas import tpu_sc as plsc

_SC_CORES = 2
_SC_SUBCORES = 16
_SC_WORKERS = _SC_CORES * _SC_SUBCORES
_SC_LANES = 16

_QB = 2048
_KC = 128


def _top2_body(q_ref, k_ref, i1_ref, i2_ref, ga_ref, gc_ref):
    b = pl.program_id(0)
    qb = pl.program_id(1)
    n = k_ref.shape[2]
    x_q = q_ref[0, :, 0:1]
    y_q = q_ref[0, :, 1:2]
    z_q = q_ref[0, :, 2:3]

    lane = lax.broadcasted_iota(jnp.int32, (_QB, _KC), 1)
    delta = (lax.broadcasted_iota(jnp.int32, (_QB, _KC), 0) + qb * _QB) - lane
    inf = jnp.float32(jnp.inf)
    m1 = jnp.full((_QB, _KC), inf)
    m2 = jnp.full((_QB, _KC), inf)
    nb = n // _KC
    c1 = jnp.full((_QB, _KC), nb, jnp.int32)
    c2 = jnp.full((_QB, _KC), nb, jnp.int32)
    for c in range(nb):
        x_k = k_ref[0, 0:1, pl.ds(c * _KC, _KC)]
        y_k = k_ref[0, 1:2, pl.ds(c * _KC, _KC)]
        z_k = k_ref[0, 2:3, pl.ds(c * _KC, _KC)]
        dx = x_q - x_k
        d2 = dx * dx
        dy = y_q - y_k
        d2 = d2 + dy * dy
        dz = z_q - z_k
        d2 = d2 + dz * dz
        v = jnp.sqrt(d2)
        v = jnp.where(delta == c * _KC, jnp.float32(1e9), v)
        take1 = v < m1
        take2 = v < m2
        m2n = jnp.where(take2, v, m2)
        c2n = jnp.where(take2, c, c2)
        m2 = jnp.where(take1, m1, m2n)
        c2 = jnp.where(take1, c1, c2n)
        m1 = jnp.where(take1, v, m1)
        c1 = jnp.where(take1, c, c1)

    i1 = c1 * _KC + lane
    i2 = c2 * _KC + lane
    big1 = jnp.min(m1, axis=1, keepdims=True)
    big_i1 = jnp.min(jnp.where(m1 == big1, i1, n), axis=1, keepdims=True)
    win = i1 == big_i1
    vals2 = jnp.where(win, m2, m1)
    idx2 = jnp.where(win, i2, i1)
    big2 = jnp.min(vals2, axis=1, keepdims=True)
    big_i2 = jnp.min(jnp.where(vals2 == big2, idx2, n), axis=1, keepdims=True)

    i1_ref[0] = big_i1
    i2_ref[0] = big_i2
    base = b * n
    ga_ref[0] = big_i1 + base
    gc_ref[0] = big_i2 + base


def _tc_top2(coords, coords_k):
    bsz, n, _ = coords.shape
    grid = (bsz, n // _QB)
    ispec = pl.BlockSpec((1, _QB, 1), lambda b, q: (b, q, 0))
    ishape = jax.ShapeDtypeStruct((bsz, n, 1), jnp.int32)
    return pl.pallas_call(
        _top2_body,
        grid=grid,
        in_specs=[
            pl.BlockSpec((1, _QB, 3), lambda b, q: (b, q, 0)),
            pl.BlockSpec((1, 8, n), lambda b, q: (b, 0, 0)),
        ],
        out_specs=[ispec, ispec, ispec, ispec],
        out_shape=[ishape, ishape, ishape, ishape],
    )(coords, coords_k)


def _sc_gather(table_t, idx_a, idx_c):
    rows = idx_a.shape[1]
    per_worker = rows * 128
    v = table_t.shape[1]
    mesh = plsc.VectorSubcoreMesh(core_axis_name="c", subcore_axis_name="s")
    out_t = jax.ShapeDtypeStruct((_SC_WORKERS, 3, per_worker), jnp.float32)

    cp = pltpu.CompilerParams()
    if "needs_layout_passes" in pltpu.CompilerParams.__dataclass_fields__:
        cp = dataclasses.replace(cp, needs_layout_passes=False)

    @functools.partial(
        pl.kernel,
        mesh=mesh,
        out_type=[out_t, out_t],
        scratch_types=[
            pltpu.VMEM((3, v), jnp.float32),
            pltpu.VMEM((rows, 128), jnp.int32),
            pltpu.VMEM((rows, 128), jnp.int32),
            pltpu.VMEM((3, per_worker), jnp.float32),
            pltpu.VMEM((3, per_worker), jnp.float32),
            pltpu.SemaphoreType.DMA,
        ],
        compiler_params=cp,
    )
    def gather_kernel(tab_hbm, ia_hbm, ic_hbm, oa_hbm, oc_hbm,
                      tab_v, ia_v, ic_v, ba_v, bc_v, sem):
        wid = lax.axis_index("s") * _SC_CORES + lax.axis_index("c")
        tab_cp = pltpu.async_copy(tab_hbm, tab_v, sem)
        pltpu.sync_copy(ia_hbm.at[wid], ia_v)
        pltpu.sync_copy(ic_hbm.at[wid], ic_v)
        tab_cp.wait()
        for idx_v, buf_v in ((ia_v, ba_v), (ic_v, bc_v)):
            for r in range(rows):
                for g in range(128 // _SC_LANES):
                    iv = idx_v[r, pl.ds(g * _SC_LANES, _SC_LANES)]
                    pos = r * 128 + g * _SC_LANES
                    for c in range(3):
                        cv = jnp.full((_SC_LANES,), c, jnp.int32)
                        vals = plsc.load_gather(tab_v, [cv, iv])
                        buf_v[c, pl.ds(pos, _SC_LANES)] = vals
        pltpu.sync_copy(ba_v, oa_hbm.at[wid])
        pltpu.sync_copy(bc_v, oc_hbm.at[wid])

    return gather_kernel(table_t, idx_a, idx_c)


def kernel(coords, mask):
    del mask
    bsz, n, _ = coords.shape
    coords_k = jnp.transpose(
        jnp.pad(coords, ((0, 0), (0, 0), (0, 5))), (0, 2, 1)
    )
    i1, i2, ga, gc = _tc_top2(coords, coords_k)
    table_t = jnp.transpose(coords.reshape(bsz * n, 3))
    rows = (bsz * n) // (_SC_WORKERS * 128)
    out_a, out_c = _sc_gather(
        table_t,
        ga.reshape(_SC_WORKERS, rows, 128),
        gc.reshape(_SC_WORKERS, rows, 128),
    )
    a = jnp.transpose(out_a, (0, 2, 1)).reshape(bsz, n, 3)
    c = jnp.transpose(out_c, (0, 2, 1)).reshape(bsz, n, 3)
    return a, c, i1.reshape(bsz, n), i2.reshape(bsz, n)
